# baseline (speedup 1.0000x reference)
import functools

import jax
import jax.numpy as jnp
from jax.experimental import pallas as pl
from jax.experimental.pallas import tpu as pltpu

LAGS = (1, 2, 4, 8, 12)
MAXLAG = 12
PADL = 128
CLAMP_MIN, CLAMP_MAX = -2.5, 2.5


def _body(params_ref, v_ref, e_ref, w1_ref, w2_ref, o_ref,
          a_ref, hist_ref, h_ref, *, tb, nt, d_f, d_g):
    t = pl.program_id(0)
    p = jax.lax.rem(t, 2)
    q = 1 - p
    d = d_f + d_g
    dp = a_ref.shape[2]

    col = jax.lax.broadcasted_iota(jnp.int32, (1, dp), 1)
    m = a_ref[p] * jnp.where(col >= d_f, h_ref[p], 1.0).astype(jnp.bfloat16)
    o_ref[0, 0] = jnp.clip(
        jnp.dot(m, w2_ref[:], preferred_element_type=jnp.float32,
                precision=jax.lax.Precision.DEFAULT),
        CLAMP_MIN, CLAMP_MAX,
    )

    hist = hist_ref[q]
    taps = [hist[:, PADL - lag:PADL - lag + tb] for lag in LAGS]
    mu = ((params_ref[0] * taps[0] + params_ref[1] * taps[1])
          + (params_ref[2] * taps[2] + params_ref[3] * taps[3])
          + (params_ref[4] * taps[4] + params_ref[10]))
    ls = ((params_ref[5] * taps[0] + params_ref[6] * taps[1])
          + (params_ref[7] * taps[2] + params_ref[8] * taps[3])
          + params_ref[9] * taps[4])
    h_ref[q] = jnp.transpose(mu + jnp.exp(ls) * e_ref[0])

    v = v_ref[0]
    r = jnp.dot(v, w1_ref[:], preferred_element_type=jnp.float32,
                precision=jax.lax.Precision.DEFAULT)
    a_ref[p] = jnp.tanh(r).astype(jnp.bfloat16)
    s = jnp.transpose(r[:, d:d + 1])
    tail = jnp.where(
        jax.lax.rem(t, nt) == 0,
        jnp.broadcast_to(s[:, 0:1], (1, MAXLAG)),
        hist_ref[q, :, PADL + tb - MAXLAG:PADL + tb],
    )
    hist_ref[p, :, PADL - MAXLAG:PADL] = tail
    hist_ref[p, :, PADL:] = s


@jax.jit
def kernel(visible, W1f, W2f, W1g, W2g, w_mu, b_mu, w_ls, b_ls):
    B, T, N = visible.shape
    d_f = W1f.shape[1]
    d_g = W1g.shape[1]
    d = d_f + d_g
    dp = 48
    tb = 2048
    nt = T // tb

    eps = jax.random.normal(jax.random.key(42), (1, B, T), jnp.float32)
    eps = eps.reshape(B, 1, T) * jnp.exp(b_ls)
    w1 = jnp.concatenate([
        W1f, W1g, jnp.full((N, 1), 1.0 / N, jnp.float32),
        jnp.zeros((N, dp - d - 1), jnp.float32),
    ], axis=1)
    w2 = jnp.concatenate([
        W2f, W2g, jnp.zeros((dp - d, N), jnp.float32)
    ], axis=0)
    w2 = w2.astype(jnp.bfloat16)
    params = jnp.concatenate([
        w_mu, w_ls, b_mu[None]
    ]).astype(jnp.float32)

    ntot = B * nt
    out = pl.pallas_call(
        functools.partial(_body, tb=tb, nt=nt, d_f=d_f, d_g=d_g),
        grid=(ntot + 2,),
        in_specs=[
            pl.BlockSpec(memory_space=pltpu.SMEM),
            pl.BlockSpec(
                (1, tb, N),
                lambda t: (jnp.minimum(t, ntot - 1) // nt,
                           jnp.minimum(t, ntot - 1) % nt, 0)),
            pl.BlockSpec(
                (1, 1, tb),
                lambda t: (jnp.clip(t - 1, 0, ntot - 1) // nt, 0,
                           jnp.clip(t - 1, 0, ntot - 1) % nt)),
            pl.BlockSpec((N, dp), lambda t: (0, 0)),
            pl.BlockSpec((dp, N), lambda t: (0, 0)),
        ],
        out_specs=pl.BlockSpec(
            (1, 1, tb, N),
            lambda t: (0, jnp.maximum(t - 2, 0) // nt,
                       jnp.maximum(t - 2, 0) % nt, 0)),
        out_shape=jax.ShapeDtypeStruct((1, B, T, N), jnp.float32),
        scratch_shapes=[
            pltpu.VMEM((2, tb, dp), jnp.bfloat16),
            pltpu.VMEM((2, 1, PADL + tb), jnp.float32),
            pltpu.VMEM((2, tb, 1), jnp.float32),
        ],
        compiler_params=pltpu.CompilerParams(
            dimension_semantics=("arbitrary",),
        ),
    )(params, visible, eps, w1, w2)
    return out


# speedup vs baseline: 1.0435x; 1.0435x over previous
import functools

import jax
import jax.numpy as jnp
from jax.experimental import pallas as pl
from jax.experimental.pallas import tpu as pltpu

LAGS = (1, 2, 4, 8, 12)
MAXLAG = 12
PADL = 128
CLAMP_MIN, CLAMP_MAX = -2.5, 2.5


def _body(params_ref, v_ref, e_ref, w1_ref, w2_ref, o_ref,
          a_ref, hist_ref, h_ref, *, tb, nt, d_f, d_g):
    t = pl.program_id(0)
    p = jax.lax.rem(t, 2)
    q = 1 - p
    d = d_f + d_g
    dp = a_ref.shape[2]

    m = a_ref[p] * h_ref[p]
    o_ref[0, 0] = jnp.clip(
        jnp.dot(m, w2_ref[:], preferred_element_type=jnp.float32,
                precision=jax.lax.Precision.DEFAULT),
        CLAMP_MIN, CLAMP_MAX,
    )

    hist = hist_ref[q]
    taps = [hist[:, PADL - lag:PADL - lag + tb] for lag in LAGS]
    mu = ((params_ref[0] * taps[0] + params_ref[1] * taps[1])
          + (params_ref[2] * taps[2] + params_ref[3] * taps[3])
          + (params_ref[4] * taps[4] + params_ref[10]))
    ls = ((params_ref[5] * taps[0] + params_ref[6] * taps[1])
          + (params_ref[7] * taps[2] + params_ref[8] * taps[3])
          + params_ref[9] * taps[4])
    hcol = jnp.transpose(mu + jnp.exp(ls) * e_ref[0])
    col = jax.lax.broadcasted_iota(jnp.int32, (1, dp), 1)
    h_ref[q] = jnp.where(col >= d_f, hcol, 1.0).astype(jnp.bfloat16)

    v = v_ref[0]
    r = jnp.dot(v, w1_ref[:], preferred_element_type=jnp.float32,
                precision=jax.lax.Precision.DEFAULT)
    a_ref[p] = jnp.tanh(r).astype(jnp.bfloat16)
    s = jnp.transpose(r[:, d:d + 1])
    tail = jnp.where(
        jax.lax.rem(t, nt) == 0,
        jnp.broadcast_to(s[:, 0:1], (1, MAXLAG)),
        hist_ref[q, :, PADL + tb - MAXLAG:PADL + tb],
    )
    hist_ref[p, :, PADL - MAXLAG:PADL] = tail
    hist_ref[p, :, PADL:] = s


@jax.jit
def kernel(visible, W1f, W2f, W1g, W2g, w_mu, b_mu, w_ls, b_ls):
    B, T, N = visible.shape
    d_f = W1f.shape[1]
    d_g = W1g.shape[1]
    d = d_f + d_g
    dp = 48
    tb = 1024
    nt = T // tb

    eps = jax.random.normal(jax.random.key(42), (1, B, T), jnp.float32)
    eps = eps.reshape(B, 1, T) * jnp.exp(b_ls)
    w1 = jnp.concatenate([
        W1f, W1g, jnp.full((N, 1), 1.0 / N, jnp.float32),
        jnp.zeros((N, dp - d - 1), jnp.float32),
    ], axis=1)
    w2 = jnp.concatenate([
        W2f, W2g, jnp.zeros((dp - d, N), jnp.float32)
    ], axis=0)
    w2 = w2.astype(jnp.bfloat16)
    params = jnp.concatenate([
        w_mu, w_ls, b_mu[None]
    ]).astype(jnp.float32)

    ntot = B * nt
    out = pl.pallas_call(
        functools.partial(_body, tb=tb, nt=nt, d_f=d_f, d_g=d_g),
        grid=(ntot + 2,),
        in_specs=[
            pl.BlockSpec(memory_space=pltpu.SMEM),
            pl.BlockSpec(
                (1, tb, N),
                lambda t: (jnp.minimum(t, ntot - 1) // nt,
                           jnp.minimum(t, ntot - 1) % nt, 0)),
            pl.BlockSpec(
                (1, 1, tb),
                lambda t: (jnp.clip(t - 1, 0, ntot - 1) // nt, 0,
                           jnp.clip(t - 1, 0, ntot - 1) % nt)),
            pl.BlockSpec((N, dp), lambda t: (0, 0)),
            pl.BlockSpec((dp, N), lambda t: (0, 0)),
        ],
        out_specs=pl.BlockSpec(
            (1, 1, tb, N),
            lambda t: (0, jnp.maximum(t - 2, 0) // nt,
                       jnp.maximum(t - 2, 0) % nt, 0)),
        out_shape=jax.ShapeDtypeStruct((1, B, T, N), jnp.float32),
        scratch_shapes=[
            pltpu.VMEM((2, tb, dp), jnp.bfloat16),
            pltpu.VMEM((2, 1, PADL + tb), jnp.float32),
            pltpu.VMEM((2, tb, dp), jnp.bfloat16),
        ],
        compiler_params=pltpu.CompilerParams(
            dimension_semantics=("arbitrary",),
        ),
    )(params, visible, eps, w1, w2)
    return out
